# baseline (speedup 1.0000x reference)
import functools

import jax
import jax.numpy as jnp
from jax.experimental import pallas as pl
from jax.experimental.pallas import tpu as pltpu

N = 50000
E = 800000
B = 512
DF = 78
H = 300
HP = 384
FF = 1024
OUT = 128

CH = 4000
NCH = E // CH
NBUF = 32
PBM = 5000
NPB = N // PBM
MBM = 2000


def _mm_body(x_ref, w_ref, b_ref, o_ref, *, relu):
    y = jnp.dot(x_ref[...], w_ref[...], preferred_element_type=jnp.float32)
    y = y + b_ref[...]
    if relu:
        y = jnp.maximum(y, 0.0)
    o_ref[...] = y


def _mm(x, w, b, bm, relu):
    m, k = x.shape
    n = w.shape[1]
    grid = (m // bm,)
    return pl.pallas_call(
        functools.partial(_mm_body, relu=relu),
        grid=grid,
        in_specs=[
            pl.BlockSpec((bm, k), lambda i: (i, 0)),
            pl.BlockSpec((k, n), lambda i: (0, 0)),
            pl.BlockSpec((1, n), lambda i: (0, 0)),
        ],
        out_specs=pl.BlockSpec((bm, n), lambda i: (i, 0)),
        out_shape=jax.ShapeDtypeStruct((m, n), jnp.float32),
    )(x, w, b)


def _degnorm_body(src_ref, dst_ref, norm_ref, dinv_ref, deg_ref):
    p = pl.program_id(0)
    c = pl.program_id(1)

    @pl.when(jnp.logical_and(p == 0, c == 0))
    def _():
        deg_ref[...] = jnp.ones_like(deg_ref)

    @pl.when(p == 0)
    def _():
        def body(i, carry):
            d = dst_ref[0, 0, i]
            deg_ref[pl.ds(d, 1), :] += 1.0
            return carry

        jax.lax.fori_loop(0, CH, body, 0)

    @pl.when(jnp.logical_and(p == 1, c == 0))
    def _():
        dinv_ref[...] = jax.lax.rsqrt(deg_ref[...])

    @pl.when(p == 1)
    def _():
        def body(i, carry):
            s = src_ref[0, 0, i]
            d = dst_ref[0, 0, i]
            norm_ref[pl.ds(i, 1), :] = (
                dinv_ref[pl.ds(s, 1), :] * dinv_ref[pl.ds(d, 1), :]
            )
            return carry

        jax.lax.fori_loop(0, CH, body, 0)


def _degnorm(src3, dst3):
    return pl.pallas_call(
        _degnorm_body,
        grid=(2, NCH),
        in_specs=[
            pl.BlockSpec((1, 1, CH), lambda p, c: (c, 0, 0),
                         memory_space=pltpu.SMEM),
            pl.BlockSpec((1, 1, CH), lambda p, c: (c, 0, 0),
                         memory_space=pltpu.SMEM),
        ],
        out_specs=[
            pl.BlockSpec((CH, 1), lambda p, c: (c, 0)),
            pl.BlockSpec((N, 1), lambda p, c: (0, 0)),
        ],
        out_shape=[
            jax.ShapeDtypeStruct((E, 1), jnp.float32),
            jax.ShapeDtypeStruct((N, 1), jnp.float32),
        ],
        scratch_shapes=[pltpu.VMEM((N, 1), jnp.float32)],
    )(src3, dst3)


HC = HP // 2


HC = HP // 2


def _agg_body(src_ref, dst_ref, norm_ref, h_hbm, o_ref, buf, sem):
    c = pl.program_id(0)

    @pl.when(c == 0)
    def _():
        o_ref[...] = jnp.zeros_like(o_ref)

    def mk(j, slot):
        s = src_ref[0, 0, j]
        return pltpu.make_async_copy(
            h_hbm.at[pl.ds(s, 1), :], buf.at[slot], sem.at[slot]
        )

    for k in range(NBUF - 1):
        mk(k, k).start()

    def body(i, carry):
        j = i + NBUF - 1

        @pl.when(j < CH)
        def _():
            mk(j, jax.lax.rem(j, NBUF)).start()

        slot = jax.lax.rem(i, NBUF)
        mk(i, slot).wait()
        d = dst_ref[0, 0, i]
        row = buf[pl.ds(slot, 1)].reshape(1, HC)
        o_ref[pl.ds(d, 1), :] += norm_ref[pl.ds(i, 1), :] * row
        return carry

    jax.lax.fori_loop(0, CH, body, 0)


def _agg_half(src3, dst3, norm, h_half):
    return pl.pallas_call(
        _agg_body,
        grid=(NCH,),
        in_specs=[
            pl.BlockSpec((1, 1, CH), lambda c: (c, 0, 0),
                         memory_space=pltpu.SMEM),
            pl.BlockSpec((1, 1, CH), lambda c: (c, 0, 0),
                         memory_space=pltpu.SMEM),
            pl.BlockSpec((CH, 1), lambda c: (c, 0)),
            pl.BlockSpec(memory_space=pl.ANY),
        ],
        out_specs=pl.BlockSpec((N, HC), lambda c: (0, 0)),
        out_shape=jax.ShapeDtypeStruct((N, HC), jnp.float32),
        scratch_shapes=[
            pltpu.VMEM((NBUF, 1, HC), jnp.float32),
            pltpu.SemaphoreType.DMA((NBUF,)),
        ],
    )(src3, dst3, norm, h_half)


def _fix_body(a_ref, h_ref, dinv_ref, b_ref, o_ref):
    di = dinv_ref[...]
    o_ref[...] = jnp.maximum(
        a_ref[...] + di * di * h_ref[...] + b_ref[...], 0.0
    )


def _fix(a, h, dinv, b):
    return pl.pallas_call(
        _fix_body,
        grid=(N // MBM,),
        in_specs=[
            pl.BlockSpec((MBM, HP), lambda i: (i, 0)),
            pl.BlockSpec((MBM, HP), lambda i: (i, 0)),
            pl.BlockSpec((MBM, 1), lambda i: (i, 0)),
            pl.BlockSpec((1, HP), lambda i: (0, 0)),
        ],
        out_specs=pl.BlockSpec((MBM, HP), lambda i: (i, 0)),
        out_shape=jax.ShapeDtypeStruct((N, HP), jnp.float32),
    )(a, h, dinv, b)


def _pool_body(batch_ref, x_ref, o_ref):
    c = pl.program_id(0)

    @pl.when(c == 0)
    def _():
        o_ref[...] = jnp.zeros_like(o_ref)

    def body(i, carry):
        g = batch_ref[0, 0, i]
        o_ref[pl.ds(g, 1), :] = jnp.maximum(
            o_ref[pl.ds(g, 1), :], x_ref[pl.ds(i, 1), :]
        )
        return carry

    jax.lax.fori_loop(0, PBM, body, 0)


def _pool(batch3, x):
    return pl.pallas_call(
        _pool_body,
        grid=(NPB,),
        in_specs=[
            pl.BlockSpec((1, 1, PBM), lambda c: (c, 0, 0),
                         memory_space=pltpu.SMEM),
            pl.BlockSpec((PBM, HP), lambda c: (c, 0)),
        ],
        out_specs=pl.BlockSpec((B, HP), lambda c: (0, 0)),
        out_shape=jax.ShapeDtypeStruct((B, HP), jnp.float32),
    )(batch3, x)


def kernel(x, edge_index, batch, W1, b1, W2, b2, W3, b3, W4, b4):
    src3 = edge_index[0].reshape(NCH, 1, CH)
    dst3 = edge_index[1].reshape(NCH, 1, CH)
    batch3 = batch.reshape(NPB, 1, PBM)

    w1p = jnp.pad(W1, ((0, 0), (0, HP - H)))
    b1p = jnp.pad(b1, (0, HP - H)).reshape(1, HP)
    w2p = jnp.pad(W2, ((0, HP - H), (0, HP - H)))
    b2p = jnp.pad(b2, (0, HP - H)).reshape(1, HP)
    w3p = jnp.pad(W3, ((0, HP - H), (0, 0)))
    zb = jnp.zeros((1, HP), jnp.float32)

    norm, dinv = _degnorm(src3, dst3)

    def agg(h):
        a0 = _agg_half(src3, dst3, norm, h[:, :HC])
        a1 = _agg_half(src3, dst3, norm, h[:, HC:])
        return jnp.concatenate([a0, a1], axis=1)

    h1 = _mm(x, w1p, zb, MBM, relu=False)
    x1 = _fix(agg(h1), h1, dinv, b1p)

    h2 = _mm(x1, w2p, zb, MBM, relu=False)
    x2 = _fix(agg(h2), h2, dinv, b2p)

    pooled = _pool(batch3, x2)

    y = _mm(pooled, w3p, b3.reshape(1, FF), B, relu=True)
    out = _mm(y, W4, b4.reshape(1, OUT), B, relu=True)
    return out


# speedup vs baseline: 4.0621x; 4.0621x over previous
import functools

import jax
import jax.numpy as jnp
from jax.experimental import pallas as pl
from jax.experimental.pallas import tpu as pltpu

N = 50000
E = 800000
B = 512
DF = 78
H = 300
HP = 384
FF = 1024
OUT = 128

CH = 4000
NCH = E // CH
NBUF = 32
PBM = 5000
NPB = N // PBM
MBM = 2000


def _mm_body(x_ref, w_ref, b_ref, o_ref, *, relu):
    y = jnp.dot(x_ref[...], w_ref[...], preferred_element_type=jnp.float32)
    y = y + b_ref[...]
    if relu:
        y = jnp.maximum(y, 0.0)
    o_ref[...] = y


def _mm(x, w, b, bm, relu):
    m, k = x.shape
    n = w.shape[1]
    grid = (m // bm,)
    return pl.pallas_call(
        functools.partial(_mm_body, relu=relu),
        grid=grid,
        in_specs=[
            pl.BlockSpec((bm, k), lambda i: (i, 0)),
            pl.BlockSpec((k, n), lambda i: (0, 0)),
            pl.BlockSpec((1, n), lambda i: (0, 0)),
        ],
        out_specs=pl.BlockSpec((bm, n), lambda i: (i, 0)),
        out_shape=jax.ShapeDtypeStruct((m, n), jnp.float32),
    )(x, w, b)


def _degnorm_body(src_ref, dst_ref, norm_ref, dinv_ref, deg_ref):
    p = pl.program_id(0)
    c = pl.program_id(1)

    @pl.when(jnp.logical_and(p == 0, c == 0))
    def _():
        deg_ref[...] = jnp.ones_like(deg_ref)

    @pl.when(p == 0)
    def _():
        def body(i, carry):
            d = dst_ref[0, 0, i]
            deg_ref[pl.ds(d, 1), :] += 1.0
            return carry

        jax.lax.fori_loop(0, CH, body, 0)

    @pl.when(jnp.logical_and(p == 1, c == 0))
    def _():
        dinv_ref[...] = jax.lax.rsqrt(deg_ref[...])

    @pl.when(p == 1)
    def _():
        def body(i, carry):
            s = src_ref[0, 0, i]
            d = dst_ref[0, 0, i]
            norm_ref[pl.ds(i, 1), :] = (
                dinv_ref[pl.ds(s, 1), :] * dinv_ref[pl.ds(d, 1), :]
            )
            return carry

        jax.lax.fori_loop(0, CH, body, 0)


def _degnorm(src3, dst3):
    return pl.pallas_call(
        _degnorm_body,
        grid=(2, NCH),
        in_specs=[
            pl.BlockSpec((1, 1, CH), lambda p, c: (c, 0, 0),
                         memory_space=pltpu.SMEM),
            pl.BlockSpec((1, 1, CH), lambda p, c: (c, 0, 0),
                         memory_space=pltpu.SMEM),
        ],
        out_specs=[
            pl.BlockSpec((CH, 1), lambda p, c: (c, 0)),
            pl.BlockSpec((N, 1), lambda p, c: (0, 0)),
        ],
        out_shape=[
            jax.ShapeDtypeStruct((E, 1), jnp.float32),
            jax.ShapeDtypeStruct((N, 1), jnp.float32),
        ],
        scratch_shapes=[pltpu.VMEM((N, 1), jnp.float32)],
    )(src3, dst3)


HC = HP // 2


HC = HP // 3


def _agg_body(src_ref, dst_ref, norm_ref, h_ref, o_ref, stage):
    c = pl.program_id(0)

    @pl.when(c == 0)
    def _():
        o_ref[...] = jnp.zeros_like(o_ref)

    def body(g, carry):
        base = g * 8
        nv = norm_ref[pl.ds(base, 8), :]
        for k in range(8):
            s = src_ref[0, 0, base + k]
            stage[pl.ds(k, 1), :] = h_ref[pl.ds(s, 1), :]
        msgs = stage[...] * nv
        for k in range(8):
            d = dst_ref[0, 0, base + k]
            o_ref[pl.ds(d, 1), :] += msgs[k:k + 1, :]
        return carry

    jax.lax.fori_loop(0, CH // 8, body, 0)


def _agg_half(src3, dst3, norm, h_half):
    return pl.pallas_call(
        _agg_body,
        grid=(NCH,),
        in_specs=[
            pl.BlockSpec((1, 1, CH), lambda c: (c, 0, 0),
                         memory_space=pltpu.SMEM),
            pl.BlockSpec((1, 1, CH), lambda c: (c, 0, 0),
                         memory_space=pltpu.SMEM),
            pl.BlockSpec((CH, 1), lambda c: (c, 0)),
            pl.BlockSpec((N, HC), lambda c: (0, 0)),
        ],
        out_specs=pl.BlockSpec((N, HC), lambda c: (0, 0)),
        out_shape=jax.ShapeDtypeStruct((N, HC), jnp.float32),
        scratch_shapes=[pltpu.VMEM((8, HC), jnp.float32)],
    )(src3, dst3, norm, h_half)


def _fix_body(a_ref, h_ref, dinv_ref, b_ref, o_ref):
    di = dinv_ref[...]
    o_ref[...] = jnp.maximum(
        a_ref[...] + di * di * h_ref[...] + b_ref[...], 0.0
    )


def _fix(a, h, dinv, b):
    return pl.pallas_call(
        _fix_body,
        grid=(N // MBM,),
        in_specs=[
            pl.BlockSpec((MBM, HP), lambda i: (i, 0)),
            pl.BlockSpec((MBM, HP), lambda i: (i, 0)),
            pl.BlockSpec((MBM, 1), lambda i: (i, 0)),
            pl.BlockSpec((1, HP), lambda i: (0, 0)),
        ],
        out_specs=pl.BlockSpec((MBM, HP), lambda i: (i, 0)),
        out_shape=jax.ShapeDtypeStruct((N, HP), jnp.float32),
    )(a, h, dinv, b)


def _pool_body(batch_ref, x_ref, o_ref):
    c = pl.program_id(0)

    @pl.when(c == 0)
    def _():
        o_ref[...] = jnp.zeros_like(o_ref)

    def body(i, carry):
        g = batch_ref[0, 0, i]
        o_ref[pl.ds(g, 1), :] = jnp.maximum(
            o_ref[pl.ds(g, 1), :], x_ref[pl.ds(i, 1), :]
        )
        return carry

    jax.lax.fori_loop(0, PBM, body, 0)


def _pool(batch3, x):
    return pl.pallas_call(
        _pool_body,
        grid=(NPB,),
        in_specs=[
            pl.BlockSpec((1, 1, PBM), lambda c: (c, 0, 0),
                         memory_space=pltpu.SMEM),
            pl.BlockSpec((PBM, HP), lambda c: (c, 0)),
        ],
        out_specs=pl.BlockSpec((B, HP), lambda c: (0, 0)),
        out_shape=jax.ShapeDtypeStruct((B, HP), jnp.float32),
    )(batch3, x)


def kernel(x, edge_index, batch, W1, b1, W2, b2, W3, b3, W4, b4):
    src3 = edge_index[0].reshape(NCH, 1, CH)
    dst3 = edge_index[1].reshape(NCH, 1, CH)
    batch3 = batch.reshape(NPB, 1, PBM)

    w1p = jnp.pad(W1, ((0, 0), (0, HP - H)))
    b1p = jnp.pad(b1, (0, HP - H)).reshape(1, HP)
    w2p = jnp.pad(W2, ((0, HP - H), (0, HP - H)))
    b2p = jnp.pad(b2, (0, HP - H)).reshape(1, HP)
    w3p = jnp.pad(W3, ((0, HP - H), (0, 0)))
    zb = jnp.zeros((1, HP), jnp.float32)

    norm, dinv = _degnorm(src3, dst3)

    def agg(h):
        parts = [
            _agg_half(src3, dst3, norm, h[:, q * HC:(q + 1) * HC])
            for q in range(HP // HC)
        ]
        return jnp.concatenate(parts, axis=1)

    h1 = _mm(x, w1p, zb, MBM, relu=False)
    x1 = _fix(agg(h1), h1, dinv, b1p)

    h2 = _mm(x1, w2p, zb, MBM, relu=False)
    x2 = _fix(agg(h2), h2, dinv, b2p)

    pooled = _pool(batch3, x2)

    y = _mm(pooled, w3p, b3.reshape(1, FF), B, relu=True)
    out = _mm(y, W4, b4.reshape(1, OUT), B, relu=True)
    return out
